# baseline (speedup 1.0000x reference)
import jax
import jax.numpy as jnp
from jax.experimental import pallas as pl
from jax.experimental.pallas import tpu as pltpu

_BLOCK = 640


def _sage_kernel(src_ref, nbr_ref, raw_ref, w_ref, b_ref, out_ref):
    k = nbr_ref.shape[1]
    aggr = jnp.sum(nbr_ref[...], axis=1) * (1.0 / k)
    neighbor_hidden = jnp.dot(aggr, w_ref[...],
                              preferred_element_type=jnp.float32)
    self_hidden = jnp.dot(src_ref[...], b_ref[...],
                          preferred_element_type=jnp.float32)
    hidden = neighbor_hidden + self_hidden
    h = hidden.shape[1]
    out_ref[:, :h] = jnp.maximum(hidden, 0.0)
    out_ref[:, h:] = jnp.maximum(raw_ref[...], 0.0)


def kernel(src_node_features, neighbor_node_features, raw_data, W, b):
    n, k, d = neighbor_node_features.shape
    h = W.shape[1]
    blk = _BLOCK
    grid = (pl.cdiv(n, blk),)
    return pl.pallas_call(
        _sage_kernel,
        grid=grid,
        in_specs=[
            pl.BlockSpec((blk, d), lambda i: (i, 0)),
            pl.BlockSpec((blk, k, d), lambda i: (i, 0, 0)),
            pl.BlockSpec((blk, d), lambda i: (i, 0)),
            pl.BlockSpec((d, h), lambda i: (0, 0)),
            pl.BlockSpec((d, h), lambda i: (0, 0)),
        ],
        out_specs=pl.BlockSpec((blk, 2 * h), lambda i: (i, 0)),
        out_shape=jax.ShapeDtypeStruct((n, 2 * h), jnp.float32),
    )(src_node_features, neighbor_node_features, raw_data, W, b)


# speedup vs baseline: 1.0075x; 1.0075x over previous
import jax
import jax.numpy as jnp
from jax.experimental import pallas as pl
from jax.experimental.pallas import tpu as pltpu

_BLOCK = 512


def _sage_kernel(src_ref, nbr_ref, raw_ref, w_ref, b_ref, out_ref):
    k = nbr_ref.shape[1]
    aggr = jnp.sum(nbr_ref[...], axis=1) * (1.0 / k)
    neighbor_hidden = jnp.dot(aggr, w_ref[...],
                              preferred_element_type=jnp.float32)
    self_hidden = jnp.dot(src_ref[...], b_ref[...],
                          preferred_element_type=jnp.float32)
    hidden = neighbor_hidden + self_hidden
    h = hidden.shape[1]
    out_ref[:, :h] = jnp.maximum(hidden, 0.0)
    out_ref[:, h:] = jnp.maximum(raw_ref[...], 0.0)


def kernel(src_node_features, neighbor_node_features, raw_data, W, b):
    n, k, d = neighbor_node_features.shape
    h = W.shape[1]
    blk = _BLOCK
    grid = (pl.cdiv(n, blk),)
    return pl.pallas_call(
        _sage_kernel,
        grid=grid,
        in_specs=[
            pl.BlockSpec((blk, d), lambda i: (i, 0)),
            pl.BlockSpec((blk, k, d), lambda i: (i, 0, 0)),
            pl.BlockSpec((blk, d), lambda i: (i, 0)),
            pl.BlockSpec((d, h), lambda i: (0, 0)),
            pl.BlockSpec((d, h), lambda i: (0, 0)),
        ],
        out_specs=pl.BlockSpec((blk, 2 * h), lambda i: (i, 0)),
        out_shape=jax.ShapeDtypeStruct((n, 2 * h), jnp.float32),
    )(src_node_features, neighbor_node_features, raw_data, W, b)
